# baseline (speedup 1.0000x reference)
import functools

import jax
import jax.numpy as jnp
from jax import lax
from jax.experimental import pallas as pl
from jax.experimental.pallas import tpu as pltpu

T = 2048
D = 768
E = 8
K = 2
H = 3072

M = 256
HC = 512
NP = 6144
NT = NP // M
NH = H // HC


def _router_body(x_ref, wr_ref, br_ref, e0_ref, e1_ref, p0_ref, p1_ref):
    logits = lax.dot_general(
        x_ref[...], wr_ref[...], (((1,), (1,)), ((), ())),
        preferred_element_type=jnp.float32,
    ) + br_ref[...][None, :]
    iota_e = lax.broadcasted_iota(jnp.int32, (T, E), 1)
    m0 = jnp.max(logits, axis=1, keepdims=True)
    i0 = jnp.min(jnp.where(logits == m0, iota_e, E), axis=1, keepdims=True)
    masked = jnp.where(iota_e == i0, -jnp.inf, logits)
    m1 = jnp.max(masked, axis=1, keepdims=True)
    i1 = jnp.min(jnp.where(masked == m1, iota_e, E), axis=1, keepdims=True)
    p0 = 1.0 / (1.0 + jnp.exp(m1 - m0))
    e0_ref[...] = i0
    e1_ref[...] = i1
    p0_ref[...] = p0
    p1_ref[...] = 1.0 - p0


def _router(x, Wr, br):
    e0, e1, p0, p1 = pl.pallas_call(
        _router_body,
        out_shape=(
            jax.ShapeDtypeStruct((T, 1), jnp.int32),
            jax.ShapeDtypeStruct((T, 1), jnp.int32),
            jax.ShapeDtypeStruct((T, 1), jnp.float32),
            jax.ShapeDtypeStruct((T, 1), jnp.float32),
        ),
    )(x, Wr, br)
    return e0[:, 0], e1[:, 0], p0[:, 0], p1[:, 0]


def _gmm_body(te_ref, xs_ref, wg_ref, bg_ref, wu_ref, bu_ref, wd_ref, bd_ref,
              y_ref):
    t = pl.program_id(0)
    h = pl.program_id(1)

    @pl.when(h == 0)
    def _():
        y_ref[...] = jnp.broadcast_to(bd_ref[0], (M, D))

    @pl.when(te_ref[t] >= 0)
    def _():
        xb = xs_ref[...]
        g = lax.dot_general(xb, wg_ref[0], (((1,), (1,)), ((), ())),
                            preferred_element_type=jnp.float32)
        g = g + bg_ref[0]
        g = g * jax.nn.sigmoid(g)
        u = lax.dot_general(xb, wu_ref[0], (((1,), (1,)), ((), ())),
                            preferred_element_type=jnp.float32)
        u = u + bu_ref[0]
        h_act = (u * g).astype(jnp.bfloat16)
        y_ref[...] += lax.dot_general(h_act, wd_ref[0],
                                      (((1,), (1,)), ((), ())),
                                      preferred_element_type=jnp.float32)


def _gmm(xs, te, Wg, bg, Wu, bu, Wd, bd):
    grid_spec = pltpu.PrefetchScalarGridSpec(
        num_scalar_prefetch=1,
        grid=(NT, NH),
        in_specs=[
            pl.BlockSpec((M, D), lambda t, h, te: (t, 0)),
            pl.BlockSpec((1, HC, D),
                         lambda t, h, te: (jnp.maximum(te[t], 0), h, 0)),
            pl.BlockSpec((1, 1, HC),
                         lambda t, h, te: (jnp.maximum(te[t], 0) * NH + h, 0, 0)),
            pl.BlockSpec((1, HC, D),
                         lambda t, h, te: (jnp.maximum(te[t], 0), h, 0)),
            pl.BlockSpec((1, 1, HC),
                         lambda t, h, te: (jnp.maximum(te[t], 0) * NH + h, 0, 0)),
            pl.BlockSpec((1, D, HC),
                         lambda t, h, te: (jnp.maximum(te[t], 0), 0, h)),
            pl.BlockSpec((1, 1, D),
                         lambda t, h, te: (jnp.maximum(te[t], 0), 0, 0)),
        ],
        out_specs=pl.BlockSpec((M, D), lambda t, h, te: (t, 0)),
    )
    return pl.pallas_call(
        _gmm_body,
        grid_spec=grid_spec,
        out_shape=jax.ShapeDtypeStruct((NP, D), jnp.float32),
        compiler_params=pltpu.CompilerParams(
            dimension_semantics=("arbitrary", "arbitrary"),
        ),
    )(te, xs, Wg.astype(jnp.bfloat16), bg.reshape(E * NH, 1, HC),
      Wu.astype(jnp.bfloat16), bu.reshape(E * NH, 1, HC),
      Wd.astype(jnp.bfloat16), bd.reshape(E, 1, D))


def kernel(x, Wr, br, Wg, bg, Wu, bu, Wd, bd):
    e0, e1, p0, p1 = _router(x, Wr, br)

    ef = jnp.concatenate([e0, e1])
    order = jnp.argsort(ef, stable=True)
    cnt = jnp.zeros((E,), jnp.int32).at[ef].add(1)
    cum = jnp.cumsum(cnt) - cnt
    pe = ((cnt + M - 1) // M) * M
    base = jnp.cumsum(pe) - pe
    used = jnp.sum(pe)
    es = ef[order]
    slot = base[es] + (jnp.arange(2 * T, dtype=jnp.int32) - cum[es])
    perm = jnp.zeros((NP,), jnp.int32).at[slot].set(
        (order % T).astype(jnp.int32))
    posm = jnp.zeros((2 * T,), jnp.int32).at[order].set(slot.astype(jnp.int32))
    pos0, pos1 = posm[:T], posm[T:]
    m_starts = jnp.arange(NT, dtype=jnp.int32) * M
    te = jnp.sum((m_starts[:, None] >= base[None, :]).astype(jnp.int32),
                 axis=1) - 1
    te = jnp.where(m_starts >= used, -1, te).astype(jnp.int32)

    xs = x.astype(jnp.bfloat16)[perm]
    ys = _gmm(xs, te, Wg, bg, Wu, bu, Wd, bd)
    out = p0[:, None] * ys[pos0] + p1[:, None] * ys[pos1]
    return out


# speedup vs baseline: 1.0954x; 1.0954x over previous
import functools

import jax
import jax.numpy as jnp
from jax import lax
from jax.experimental import pallas as pl
from jax.experimental.pallas import tpu as pltpu

T = 2048
D = 768
E = 8
K = 2
H = 3072

M = 256
HC = 512
NP = 6144
NT = NP // M
NH = H // HC


def _router_body(x_ref, wr_ref, br_ref, e0_ref, e1_ref, p0_ref, p1_ref):
    logits = lax.dot_general(
        x_ref[...], wr_ref[...], (((1,), (1,)), ((), ())),
        preferred_element_type=jnp.float32,
    ) + br_ref[...][None, :]
    iota_e = lax.broadcasted_iota(jnp.int32, (T, E), 1)
    m0 = jnp.max(logits, axis=1, keepdims=True)
    i0 = jnp.min(jnp.where(logits == m0, iota_e, E), axis=1, keepdims=True)
    masked = jnp.where(iota_e == i0, -jnp.inf, logits)
    m1 = jnp.max(masked, axis=1, keepdims=True)
    i1 = jnp.min(jnp.where(masked == m1, iota_e, E), axis=1, keepdims=True)
    p0 = 1.0 / (1.0 + jnp.exp(m1 - m0))
    e0_ref[...] = i0
    e1_ref[...] = i1
    p0_ref[...] = p0
    p1_ref[...] = 1.0 - p0


def _router(x, Wr, br):
    e0, e1, p0, p1 = pl.pallas_call(
        _router_body,
        out_shape=(
            jax.ShapeDtypeStruct((T, 1), jnp.int32),
            jax.ShapeDtypeStruct((T, 1), jnp.int32),
            jax.ShapeDtypeStruct((T, 1), jnp.float32),
            jax.ShapeDtypeStruct((T, 1), jnp.float32),
        ),
    )(x, Wr, br)
    return e0[:, 0], e1[:, 0], p0[:, 0], p1[:, 0]


def _gmm_body(te_ref, xs_ref, wg_ref, bg_ref, wu_ref, bu_ref, wd_ref, bd_ref,
              y_ref):
    t = pl.program_id(0)
    h = pl.program_id(1)

    @pl.when(h == 0)
    def _():
        y_ref[...] = jnp.broadcast_to(bd_ref[0], (M, D))

    @pl.when(te_ref[t] >= 0)
    def _():
        xb = xs_ref[...].astype(jnp.bfloat16)
        g = lax.dot_general(xb, wg_ref[0].astype(jnp.bfloat16),
                            (((1,), (1,)), ((), ())),
                            preferred_element_type=jnp.float32)
        g = g + bg_ref[0]
        g = g * jax.nn.sigmoid(g)
        u = lax.dot_general(xb, wu_ref[0].astype(jnp.bfloat16),
                            (((1,), (1,)), ((), ())),
                            preferred_element_type=jnp.float32)
        u = u + bu_ref[0]
        h_act = (u * g).astype(jnp.bfloat16)
        y_ref[...] += lax.dot_general(h_act, wd_ref[0].astype(jnp.bfloat16),
                                      (((1,), (1,)), ((), ())),
                                      preferred_element_type=jnp.float32)


def _gmm(xs, te, Wg, bg, Wu, bu, Wd, bd):
    grid_spec = pltpu.PrefetchScalarGridSpec(
        num_scalar_prefetch=1,
        grid=(NT, NH),
        in_specs=[
            pl.BlockSpec((M, D), lambda t, h, te: (t, 0)),
            pl.BlockSpec((1, HC, D),
                         lambda t, h, te: (jnp.maximum(te[t], 0), h, 0)),
            pl.BlockSpec((1, 1, HC),
                         lambda t, h, te: (jnp.maximum(te[t], 0) * NH + h, 0, 0)),
            pl.BlockSpec((1, HC, D),
                         lambda t, h, te: (jnp.maximum(te[t], 0), h, 0)),
            pl.BlockSpec((1, 1, HC),
                         lambda t, h, te: (jnp.maximum(te[t], 0) * NH + h, 0, 0)),
            pl.BlockSpec((1, D, HC),
                         lambda t, h, te: (jnp.maximum(te[t], 0), 0, h)),
            pl.BlockSpec((1, 1, D),
                         lambda t, h, te: (jnp.maximum(te[t], 0), 0, 0)),
        ],
        out_specs=pl.BlockSpec((M, D), lambda t, h, te: (t, 0)),
    )
    return pl.pallas_call(
        _gmm_body,
        grid_spec=grid_spec,
        out_shape=jax.ShapeDtypeStruct((NP, D), jnp.float32),
        compiler_params=pltpu.CompilerParams(
            dimension_semantics=("arbitrary", "arbitrary"),
        ),
    )(te, xs, Wg, bg.reshape(E * NH, 1, HC), Wu, bu.reshape(E * NH, 1, HC),
      Wd, bd.reshape(E, 1, D))


def kernel(x, Wr, br, Wg, bg, Wu, bu, Wd, bd):
    e0, e1, p0, p1 = _router(x, Wr, br)

    ef = jnp.concatenate([e0, e1])
    order = jnp.argsort(ef, stable=True)
    cnt = jnp.zeros((E,), jnp.int32).at[ef].add(1)
    cum = jnp.cumsum(cnt) - cnt
    pe = ((cnt + M - 1) // M) * M
    base = jnp.cumsum(pe) - pe
    used = jnp.sum(pe)
    es = ef[order]
    slot = base[es] + (jnp.arange(2 * T, dtype=jnp.int32) - cum[es])
    perm = jnp.zeros((NP,), jnp.int32).at[slot].set(
        (order % T).astype(jnp.int32))
    posm = jnp.zeros((2 * T,), jnp.int32).at[order].set(slot.astype(jnp.int32))
    pos0, pos1 = posm[:T], posm[T:]
    m_starts = jnp.arange(NT, dtype=jnp.int32) * M
    te = jnp.sum((m_starts[:, None] >= base[None, :]).astype(jnp.int32),
                 axis=1) - 1
    te = jnp.where(m_starts >= used, -1, te).astype(jnp.int32)

    xs = x[perm]
    ys = _gmm(xs, te, Wg, bg, Wu, bu, Wd, bd)
    out = p0[:, None] * ys[pos0] + p1[:, None] * ys[pos1]
    return out


# speedup vs baseline: 1.3308x; 1.2149x over previous
import functools

import jax
import jax.numpy as jnp
from jax import lax
from jax.experimental import pallas as pl
from jax.experimental.pallas import tpu as pltpu

T = 2048
D = 768
E = 8
K = 2
H = 3072

M = 192
HC = 1536
NP = 5760
NT = NP // M
NH = H // HC


def _router_body(x_ref, wr_ref, br_ref, e0_ref, e1_ref, p0_ref, p1_ref):
    logits = lax.dot_general(
        x_ref[...], wr_ref[...], (((1,), (1,)), ((), ())),
        preferred_element_type=jnp.float32,
    ) + br_ref[...][None, :]
    iota_e = lax.broadcasted_iota(jnp.int32, (T, E), 1)
    m0 = jnp.max(logits, axis=1, keepdims=True)
    i0 = jnp.min(jnp.where(logits == m0, iota_e, E), axis=1, keepdims=True)
    masked = jnp.where(iota_e == i0, -jnp.inf, logits)
    m1 = jnp.max(masked, axis=1, keepdims=True)
    i1 = jnp.min(jnp.where(masked == m1, iota_e, E), axis=1, keepdims=True)
    p0 = 1.0 / (1.0 + jnp.exp(m1 - m0))
    e0_ref[...] = i0
    e1_ref[...] = i1
    p0_ref[...] = p0
    p1_ref[...] = 1.0 - p0


def _router(x, Wr, br):
    e0, e1, p0, p1 = pl.pallas_call(
        _router_body,
        out_shape=(
            jax.ShapeDtypeStruct((T, 1), jnp.int32),
            jax.ShapeDtypeStruct((T, 1), jnp.int32),
            jax.ShapeDtypeStruct((T, 1), jnp.float32),
            jax.ShapeDtypeStruct((T, 1), jnp.float32),
        ),
    )(x, Wr, br)
    return e0[:, 0], e1[:, 0], p0[:, 0], p1[:, 0]


def _gmm_body(te_ref, xs_ref, wg_ref, bg_ref, wu_ref, bu_ref, wd_ref, bd_ref,
              y_ref, acc_ref):
    h = pl.program_id(0)
    t = pl.program_id(1)

    @pl.when(te_ref[t] >= 0)
    def _():
        xb = xs_ref[...]
        g = lax.dot_general(xb, wg_ref[0], (((1,), (1,)), ((), ())),
                            preferred_element_type=jnp.float32)
        g = g + bg_ref[0]
        g = g * jax.nn.sigmoid(g)
        u = lax.dot_general(xb, wu_ref[0], (((1,), (1,)), ((), ())),
                            preferred_element_type=jnp.float32)
        u = u + bu_ref[0]
        contrib = lax.dot_general(u * g, wd_ref[0], (((1,), (1,)), ((), ())),
                                  preferred_element_type=jnp.float32)
        rows = pl.ds(t * M, M)

        @pl.when(h == 0)
        def _():
            acc_ref[rows, :] = contrib + bd_ref[0]

        @pl.when(h == NH - 1)
        def _():
            y_ref[...] = acc_ref[rows, :] + contrib


def _gmm(xs, te, Wg, bg, Wu, bu, Wd, bd):
    def we(te, t):
        return jnp.where(te[t] < 0, E - 1, te[t])

    grid_spec = pltpu.PrefetchScalarGridSpec(
        num_scalar_prefetch=1,
        grid=(NH, NT),
        in_specs=[
            pl.BlockSpec((M, D), lambda h, t, te: (t, 0)),
            pl.BlockSpec((1, HC, D), lambda h, t, te: (we(te, t), h, 0)),
            pl.BlockSpec((1, 1, HC),
                         lambda h, t, te: (we(te, t) * NH + h, 0, 0)),
            pl.BlockSpec((1, HC, D), lambda h, t, te: (we(te, t), h, 0)),
            pl.BlockSpec((1, 1, HC),
                         lambda h, t, te: (we(te, t) * NH + h, 0, 0)),
            pl.BlockSpec((1, D, HC), lambda h, t, te: (we(te, t), 0, h)),
            pl.BlockSpec((1, 1, D), lambda h, t, te: (we(te, t), 0, 0)),
        ],
        out_specs=pl.BlockSpec((M, D), lambda h, t, te: (t, 0)),
        scratch_shapes=[pltpu.VMEM((NP, D), jnp.float32)],
    )
    return pl.pallas_call(
        _gmm_body,
        grid_spec=grid_spec,
        out_shape=jax.ShapeDtypeStruct((NP, D), jnp.float32),
        compiler_params=pltpu.CompilerParams(
            dimension_semantics=("arbitrary", "arbitrary"),
        ),
    )(te, xs, Wg, bg.reshape(E * NH, 1, HC), Wu, bu.reshape(E * NH, 1, HC),
      Wd, bd.reshape(E, 1, D))


def kernel(x, Wr, br, Wg, bg, Wu, bu, Wd, bd):
    e0, e1, p0, p1 = _router(x, Wr, br)

    ef = jnp.concatenate([e0, e1])
    order = jnp.argsort(ef, stable=True)
    cnt = jnp.zeros((E,), jnp.int32).at[ef].add(1)
    cum = jnp.cumsum(cnt) - cnt
    pe = ((cnt + M - 1) // M) * M
    base = jnp.cumsum(pe) - pe
    used = jnp.sum(pe)
    es = ef[order]
    slot = base[es] + (jnp.arange(2 * T, dtype=jnp.int32) - cum[es])
    perm = jnp.zeros((NP,), jnp.int32).at[slot].set(
        (order % T).astype(jnp.int32))
    posm = jnp.zeros((2 * T,), jnp.int32).at[order].set(slot.astype(jnp.int32))
    pos0, pos1 = posm[:T], posm[T:]
    m_starts = jnp.arange(NT, dtype=jnp.int32) * M
    te = jnp.sum((m_starts[:, None] >= base[None, :]).astype(jnp.int32),
                 axis=1) - 1
    te = jnp.where(m_starts >= used, -1, te).astype(jnp.int32)

    xs = x[perm]
    ys = _gmm(xs, te, Wg, bg, Wu, bu, Wd, bd)
    out = p0[:, None] * ys[pos0] + p1[:, None] * ys[pos1]
    return out
